# baseline (speedup 1.0000x reference)
import functools

import jax
import jax.numpy as jnp
from jax import lax
from jax.experimental import pallas as pl
from jax.experimental.pallas import tpu as pltpu
from jax.experimental.pallas import tpu_sc as plsc

S, P, D, E = 100000, 20000, 64, 1000000
NC, NS, L = 2, 16, 16
W = 128
CPT = 492
G4 = CPT // 4
G12 = CPT // 12
NCH = NS * CPT
EPAD = NCH * W
SCHUNK = 25000
NPASS = S // SCHUNK
ZB = 32
CB = 200
PBLK = P // ZB
SBLK = -(-SCHUNK // ZB)
ACCR = SBLK * ZB
CCAP = 768


def _sc_body(student, problem, rows_h, cols_h, vals_h, zeros_h,
             out_sc, out_sic, out_pc, out_pic,
             acc, zbuf, gath, rbuf, cbuf, vbuf,
             clrow, ccol, cval, fl0, fl1,
             es0, es1, es2, es3, gs0, gs1, gs2, ss0, ss1, ss2, zsem):
    c = lax.axis_index("c")
    t = lax.axis_index("s")
    esem = [es0, es1, es2, es3]
    gsem = [gs0, gs1, gs2]
    ssem = [ss0, ss1, ss2]
    flb = [fl0, fl1]
    pltpu.sync_copy(zeros_h, zbuf)
    iota = lax.iota(jnp.int32, L)
    zvec = jnp.zeros((L,), jnp.int32)

    def prefill(i, _):
        ccol[pl.ds(i * L, L)] = zvec
        return 0

    lax.fori_loop(0, CCAP // L, prefill, 0)

    def batched_blocks(nblk, bs, fn):
        nj = -(-nblk // NS)

        def issue(j, _):
            b = t + NS * j

            @pl.when(b < nblk)
            def _():
                fn(b * bs, True)
            return 0

        def drain(j, _):
            b = t + NS * j

            @pl.when(b < nblk)
            def _():
                fn(b * bs, False)
            return 0

        lax.fori_loop(0, nj, issue, 0)
        lax.fori_loop(0, nj, drain, 0)

    def zero_acc(nblk):
        def fn(r0, issuing):
            if issuing:
                pltpu.async_copy(zbuf, acc.at[pl.ds(r0, ZB)], zsem)
            else:
                pltpu.make_async_copy(zbuf, acc.at[pl.ds(r0, ZB)], zsem).wait()
        batched_blocks(nblk, ZB, fn)

    def copy_out(nblk, base, out_ref):
        def fn(r0, issuing):
            src = acc.at[pl.ds(r0, CB)]
            dst = out_ref.at[pl.ds(base + r0, CB)]
            if issuing:
                pltpu.async_copy(src, dst, zsem)
            else:
                pltpu.make_async_copy(src, dst, zsem).wait()
        batched_blocks(nblk, CB, fn)

    def kidx(w):
        return c * NCH + t * CPT + w

    def edges(w, se, issuing):
        k = kidx(w)
        for hsrc, dst in ((rows_h, rbuf), (cols_h, cbuf), (vals_h, vbuf)):
            if issuing:
                pltpu.async_copy(hsrc.at[k], dst.at[se], esem[se])
            else:
                pltpu.make_async_copy(hsrc.at[k], dst.at[se], esem[se]).wait()

    def scale(buf_ref, val_ref, voff):
        def body(g):
            vv = val_ref[pl.ds(voff + g * L, L)]
            for p in range(L):
                vsp = jnp.full((L,), vv[p])
                e = g * L + p
                for j in range(D // L):
                    buf_ref[e, pl.ds(j * L, L)] = (
                        buf_ref[e, pl.ds(j * L, L)] * vsp)

        plsc.parallel_loop(0, W // L, 1, unroll=2)(body)

    def run_pass_a():
        def gather(es, gs, issuing):
            if issuing:
                pltpu.async_copy(student.at[rbuf.at[es]], gath.at[gs], gsem[gs])
            else:
                pltpu.make_async_copy(student.at[rbuf.at[es]], gath.at[gs],
                                      gsem[gs]).wait()

        def scatter(es, gs, issuing):
            if issuing:
                pltpu.async_copy(gath.at[gs], acc.at[cbuf.at[es]], ssem[gs],
                                 add=True)
            else:
                pltpu.make_async_copy(gath.at[gs], acc.at[cbuf.at[es]],
                                      ssem[gs]).wait()

        edges(0, 0, True)
        edges(1, 1, True)
        edges(0, 0, False)
        gather(0, 0, True)

        def loop_body(g, _):
            for u in range(12):
                w = 12 * g + u
                es, gs = u % 4, u % 3

                def mid(with_e2):
                    def drain():
                        scatter((u + 2) % 4, (u + 1) % 3, False)
                    if u >= 2:
                        drain()
                    else:
                        @pl.when(g >= 1)
                        def _():
                            drain()
                    edges(w + 1, (u + 1) % 4, False)
                    gather((u + 1) % 4, (u + 1) % 3, True)
                    if with_e2:
                        edges(w + 2, (u + 2) % 4, True)

                gather(es, gs, False)
                if u == 11:
                    @pl.when(g < G12 - 1)
                    def _():
                        mid(True)
                elif u == 10:
                    mid(False)

                    @pl.when(g < G12 - 1)
                    def _():
                        edges(w + 2, (u + 2) % 4, True)
                else:
                    mid(True)
                scale(gath.at[gs], vbuf.at[es], 0)
                scatter(es, gs, True)
            return 0

        lax.fori_loop(0, G12, loop_body, 0)
        scatter((CPT - 3) % 4, (CPT - 3) % 3, False)
        scatter((CPT - 2) % 4, (CPT - 2) % 3, False)
        scatter((CPT - 1) % 4, (CPT - 1) % 3, False)

    def drain_flush(sg):
        pltpu.make_async_copy(gath.at[sg], acc.at[flb[sg]], ssem[sg]).wait()

    def sub_flush(k, nvalid, masked, pend, wait_scatter):
        sg = k % 2
        fl = flb[sg]
        off = k * W
        if k >= 2:
            drain_flush(sg)
        else:
            @pl.when((pend & (1 << sg)) > 0)
            def _():
                drain_flush(sg)
        pltpu.async_copy(problem.at[ccol.at[pl.ds(off, W)]], gath.at[sg],
                         gsem[sg])
        for i in range(W // L):
            src = clrow[pl.ds(off + i * L, L)]
            if masked:
                mm = (iota + (off + i * L)) < nvalid
                fl[pl.ds(i * L, L)] = jnp.where(mm, src, iota * 1024 + i)
                vsrc = cval[pl.ds(off + i * L, L)]
                cval[pl.ds(off + i * L, L)] = jnp.where(mm, vsrc, 0.0)
            else:
                fl[pl.ds(i * L, L)] = src
        pltpu.make_async_copy(problem.at[ccol.at[pl.ds(off, W)]], gath.at[sg],
                              gsem[sg]).wait()
        scale(gath.at[sg], cval, off)
        pltpu.async_copy(gath.at[sg], acc.at[fl], ssem[sg], add=True)
        if wait_scatter:
            drain_flush(sg)

    def run_pass_b(base):
        def scan_window(se, cnt):
            for g in range(W // L):
                r = rbuf[se, pl.ds(g * L, L)]
                m = (r >= base) & (r < base + SCHUNK)
                plsc.store_compressed(clrow.at[pl.ds(cnt, L)], r - base, mask=m)
                plsc.store_compressed(ccol.at[pl.ds(cnt, L)],
                                      cbuf[se, pl.ds(g * L, L)], mask=m)
                plsc.store_compressed(cval.at[pl.ds(cnt, L)],
                                      vbuf[se, pl.ds(g * L, L)], mask=m)
                cnt = cnt + plsc.all_reduce_population_count(m)[0]
            return cnt

        edges(0, 0, True)
        edges(1, 1, True)

        def loop_body(g, carry):
            cnt, pend = carry
            for u in range(4):
                w = 4 * g + u
                if u >= 2:
                    @pl.when(g < G4 - 1)
                    def _():
                        edges(w + 2, (u + 2) % 4, True)
                else:
                    edges(w + 2, (u + 2) % 4, True)
                edges(w, u, False)
                cnt = scan_window(u, cnt)
            nf = cnt // W
            for k in range(4):
                @pl.when(k < nf)
                def _():
                    sub_flush(k, cnt, False, pend, False)
            pend = (pend
                    | jnp.where(nf >= 1, 1, 0)
                    | jnp.where(nf >= 2, 2, 0))
            flushed = cnt - lax.rem(cnt, W)
            for i in range(W // L):
                clrow[pl.ds(i * L, L)] = clrow[pl.ds(flushed + i * L, L)]
                ccol[pl.ds(i * L, L)] = ccol[pl.ds(flushed + i * L, L)]
                cval[pl.ds(i * L, L)] = cval[pl.ds(flushed + i * L, L)]
            return (lax.rem(cnt, W), pend)

        cnt, pend = lax.fori_loop(0, G4, loop_body, (0, 0))

        @pl.when((cnt == 0) & ((pend & 1) > 0))
        def _():
            drain_flush(0)

        @pl.when(cnt > 0)
        def _():
            sub_flush(0, cnt, True, pend, True)

        @pl.when((pend & 2) > 0)
        def _():
            drain_flush(1)

    zero_acc(PBLK)
    plsc.subcore_barrier()
    run_pass_a()
    plsc.subcore_barrier()

    @pl.when(c == 0)
    def _():
        copy_out(P // CB, 0, out_pc)

    @pl.when(c == 1)
    def _():
        copy_out(P // CB, 0, out_pic)
    plsc.subcore_barrier()

    def phase_b_pass(q, _):
        base = q * SCHUNK
        zero_acc(SBLK)
        plsc.subcore_barrier()
        run_pass_b(base)
        plsc.subcore_barrier()

        @pl.when(c == 0)
        def _():
            copy_out(SCHUNK // CB, base, out_sc)

        @pl.when(c == 1)
        def _():
            copy_out(SCHUNK // CB, base, out_sic)
        plsc.subcore_barrier()
        return 0

    lax.fori_loop(0, NPASS, phase_b_pass, 0)


@jax.jit
def kernel(student_embeds, problem_embeds, a_rows, a_cols, a_values,
           ia_rows, ia_cols, ia_values):
    npad = EPAD - E
    pad_r = (jnp.arange(npad, dtype=jnp.int32) * 97) % S
    pad_c = (jnp.arange(npad, dtype=jnp.int32) * 89) % P
    pad_v = jnp.zeros((npad,), jnp.float32)

    def prep(x, pad):
        return jnp.concatenate([x, pad]).reshape(NCH, W)

    rows_h = jnp.concatenate([prep(a_rows, pad_r), prep(ia_rows, pad_r)])
    cols_h = jnp.concatenate([prep(a_cols, pad_c), prep(ia_cols, pad_c)])
    vals_h = jnp.concatenate([prep(a_values, pad_v), prep(ia_values, pad_v)])
    zeros_h = jnp.zeros((ZB, D), jnp.float32)

    mesh = plsc.VectorSubcoreMesh(core_axis_name="c", subcore_axis_name="s",
                                  num_cores=NC, num_subcores=NS)
    out = pl.kernel(
        _sc_body,
        out_type=(
            jax.ShapeDtypeStruct((S, D), jnp.float32),
            jax.ShapeDtypeStruct((S, D), jnp.float32),
            jax.ShapeDtypeStruct((P, D), jnp.float32),
            jax.ShapeDtypeStruct((P, D), jnp.float32),
        ),
        mesh=mesh,
        compiler_params=pltpu.CompilerParams(use_tc_tiling_on_sc=False,
                                             needs_layout_passes=False),
        scratch_types=[
            pltpu.VMEM_SHARED((ACCR, D), jnp.float32),
            pltpu.VMEM((ZB, D), jnp.float32),
            pltpu.VMEM((3, W, D), jnp.float32),
            pltpu.VMEM((4, W), jnp.int32),
            pltpu.VMEM((4, W), jnp.int32),
            pltpu.VMEM((4, W), jnp.float32),
            pltpu.VMEM((CCAP,), jnp.int32),
            pltpu.VMEM((CCAP,), jnp.int32),
            pltpu.VMEM((CCAP,), jnp.float32),
            pltpu.VMEM((W,), jnp.int32),
            pltpu.VMEM((W,), jnp.int32),
        ] + [pltpu.SemaphoreType.DMA] * 11,
    )(student_embeds, problem_embeds, rows_h, cols_h, vals_h, zeros_h)
    return out


# speedup vs baseline: 1.4221x; 1.4221x over previous
import functools

import jax
import jax.numpy as jnp
from jax import lax
from jax.experimental import pallas as pl
from jax.experimental.pallas import tpu as pltpu
from jax.experimental.pallas import tpu_sc as plsc

S, P, D, E = 100000, 20000, 64, 1000000
NC, NS, L = 2, 16, 16
W = 128
CPT = 492
G4 = CPT // 4
G12 = CPT // 12
NCH = NS * CPT
EPAD = NCH * W
SCHUNK = 25000
NPASS = S // SCHUNK
ZB = 32
CB = 200
PBLK = P // ZB
SBLK = -(-SCHUNK // ZB)
ACCR = SBLK * ZB
CCAP = 768


def _sc_body(student, problem, rows_h, cols_h, vals_h, zeros_h,
             out_sc, out_sic, out_pc, out_pic,
             acc, zbuf, gath, rbuf, cbuf, vbuf,
             clrow, ccol, cval, fl0, fl1,
             es0, es1, es2, es3, gs0, gs1, gs2, ss0, ss1, ss2, zsem):
    c = lax.axis_index("c")
    t = lax.axis_index("s")
    esem = [es0, es1, es2, es3]
    gsem = [gs0, gs1, gs2]
    ssem = [ss0, ss1, ss2]
    flb = [fl0, fl1]
    pltpu.sync_copy(zeros_h, zbuf)
    iota = lax.iota(jnp.int32, L)
    zvec = jnp.zeros((L,), jnp.int32)

    def prefill(i, _):
        ccol[pl.ds(i * L, L)] = zvec
        return 0

    lax.fori_loop(0, CCAP // L, prefill, 0)

    def batched_blocks(nblk, bs, fn):
        nj = -(-nblk // NS)

        def issue(j, _):
            b = t + NS * j

            @pl.when(b < nblk)
            def _():
                fn(b * bs, True)
            return 0

        def drain(j, _):
            b = t + NS * j

            @pl.when(b < nblk)
            def _():
                fn(b * bs, False)
            return 0

        lax.fori_loop(0, nj, issue, 0)
        lax.fori_loop(0, nj, drain, 0)

    def zero_acc(nblk):
        def fn(r0, issuing):
            if issuing:
                pltpu.async_copy(zbuf, acc.at[pl.ds(r0, ZB)], zsem)
            else:
                pltpu.make_async_copy(zbuf, acc.at[pl.ds(r0, ZB)], zsem).wait()
        batched_blocks(nblk, ZB, fn)

    def copy_out(nblk, base, out_ref):
        def fn(r0, issuing):
            src = acc.at[pl.ds(r0, CB)]
            dst = out_ref.at[pl.ds(base + r0, CB)]
            if issuing:
                pltpu.async_copy(src, dst, zsem)
            else:
                pltpu.make_async_copy(src, dst, zsem).wait()
        batched_blocks(nblk, CB, fn)

    def kidx(w):
        return c * NCH + t * CPT + w

    def edges(w, se, issuing):
        k = kidx(w)
        for hsrc, dst in ((rows_h, rbuf), (cols_h, cbuf), (vals_h, vbuf)):
            if issuing:
                pltpu.async_copy(hsrc.at[k], dst.at[se], esem[se])
            else:
                pltpu.make_async_copy(hsrc.at[k], dst.at[se], esem[se]).wait()

    def scale(buf_ref, val_ref, voff):
        def body(g):
            vv = val_ref[pl.ds(voff + g * L, L)]
            for p in range(L):
                vsp = jnp.full((L,), vv[p])
                e = g * L + p
                for j in range(D // L):
                    buf_ref[e, pl.ds(j * L, L)] = (
                        buf_ref[e, pl.ds(j * L, L)] * vsp)

        plsc.parallel_loop(0, W // L, 1, unroll=2)(body)

    def run_pass_a():
        def gather(es, gs, issuing):
            if issuing:
                pltpu.async_copy(student.at[rbuf.at[es]], gath.at[gs], gsem[gs])
            else:
                pltpu.make_async_copy(student.at[rbuf.at[es]], gath.at[gs],
                                      gsem[gs]).wait()

        def scatter(es, gs, issuing):
            if issuing:
                pltpu.async_copy(gath.at[gs], acc.at[cbuf.at[es]], ssem[gs],
                                 add=True)
            else:
                pltpu.make_async_copy(gath.at[gs], acc.at[cbuf.at[es]],
                                      ssem[gs]).wait()

        edges(0, 0, True)
        edges(1, 1, True)
        edges(0, 0, False)
        gather(0, 0, True)

        def loop_body(g, _):
            for u in range(12):
                w = 12 * g + u
                es, gs = u % 4, u % 3

                def mid(with_e2):
                    def drain():
                        scatter((u + 2) % 4, (u + 1) % 3, False)
                    if u >= 2:
                        drain()
                    else:
                        @pl.when(g >= 1)
                        def _():
                            drain()
                    edges(w + 1, (u + 1) % 4, False)
                    gather((u + 1) % 4, (u + 1) % 3, True)
                    if with_e2:
                        edges(w + 2, (u + 2) % 4, True)

                gather(es, gs, False)
                if u == 11:
                    @pl.when(g < G12 - 1)
                    def _():
                        mid(True)
                elif u == 10:
                    mid(False)

                    @pl.when(g < G12 - 1)
                    def _():
                        edges(w + 2, (u + 2) % 4, True)
                else:
                    mid(True)
                scale(gath.at[gs], vbuf.at[es], 0)
                scatter(es, gs, True)
            return 0

        lax.fori_loop(0, G12, loop_body, 0)
        scatter((CPT - 3) % 4, (CPT - 3) % 3, False)
        scatter((CPT - 2) % 4, (CPT - 2) % 3, False)
        scatter((CPT - 1) % 4, (CPT - 1) % 3, False)

    def drain_flush(sg):
        pltpu.make_async_copy(gath.at[sg], acc.at[flb[sg]], ssem[sg]).wait()

    def flush_start(k, nvalid, masked, pend):
        sg = k % 2
        fl = flb[sg]
        off = k * W
        if k >= 2:
            drain_flush(sg)
        else:
            @pl.when((pend & (1 << sg)) > 0)
            def _():
                drain_flush(sg)
        pltpu.async_copy(problem.at[ccol.at[pl.ds(off, W)]], gath.at[sg],
                         gsem[sg])
        for i in range(W // L):
            src = clrow[pl.ds(off + i * L, L)]
            if masked:
                mm = (iota + (off + i * L)) < nvalid
                fl[pl.ds(i * L, L)] = jnp.where(mm, src, iota * 1024 + i)
                vsrc = cval[pl.ds(off + i * L, L)]
                cval[pl.ds(off + i * L, L)] = jnp.where(mm, vsrc, 0.0)
            else:
                fl[pl.ds(i * L, L)] = src

    def flush_finish(k, wait_scatter):
        sg = k % 2
        off = k * W
        pltpu.make_async_copy(problem.at[ccol.at[pl.ds(off, W)]], gath.at[sg],
                              gsem[sg]).wait()
        scale(gath.at[sg], cval, off)
        pltpu.async_copy(gath.at[sg], acc.at[flb[sg]], ssem[sg], add=True)
        if wait_scatter:
            drain_flush(sg)

    def sub_flush(k, nvalid, masked, pend, wait_scatter):
        flush_start(k, nvalid, masked, pend)
        flush_finish(k, wait_scatter)

    def run_pass_b(base):
        def scan_window(se, cnt):
            for g in range(W // L):
                r = rbuf[se, pl.ds(g * L, L)]
                m = (r >= base) & (r < base + SCHUNK)
                plsc.store_compressed(clrow.at[pl.ds(cnt, L)], r - base, mask=m)
                plsc.store_compressed(ccol.at[pl.ds(cnt, L)],
                                      cbuf[se, pl.ds(g * L, L)], mask=m)
                plsc.store_compressed(cval.at[pl.ds(cnt, L)],
                                      vbuf[se, pl.ds(g * L, L)], mask=m)
                cnt = cnt + plsc.all_reduce_population_count(m)[0]
            return cnt

        edges(0, 0, True)
        edges(1, 1, True)

        def loop_body(g, carry):
            cnt, pend = carry
            for u in range(4):
                w = 4 * g + u
                if u >= 2:
                    @pl.when(g < G4 - 1)
                    def _():
                        edges(w + 2, (u + 2) % 4, True)
                else:
                    edges(w + 2, (u + 2) % 4, True)
                edges(w, u, False)
                cnt = scan_window(u, cnt)
            nf = jnp.where(cnt >= 2 * W, cnt // W, 0)
            for k in range(2):
                @pl.when(k < nf)
                def _():
                    flush_start(k, cnt, False, pend)
            for k in range(2):
                @pl.when(k < nf)
                def _():
                    flush_finish(k, False)
            for k in range(2, 5):
                @pl.when(k < nf)
                def _():
                    sub_flush(k, cnt, False, pend, False)
            pend = (pend
                    | jnp.where(nf >= 1, 1, 0)
                    | jnp.where(nf >= 2, 2, 0))
            flushed = cnt - lax.rem(cnt, W)
            for i in range(W // L):
                clrow[pl.ds(i * L, L)] = clrow[pl.ds(flushed + i * L, L)]
                ccol[pl.ds(i * L, L)] = ccol[pl.ds(flushed + i * L, L)]
                cval[pl.ds(i * L, L)] = cval[pl.ds(flushed + i * L, L)]
            return (lax.rem(cnt, W), pend)

        cnt, pend = lax.fori_loop(0, G4, loop_body, (0, 0))

        @pl.when((cnt == 0) & ((pend & 1) > 0))
        def _():
            drain_flush(0)

        @pl.when(cnt > 0)
        def _():
            sub_flush(0, cnt, True, pend, True)

        @pl.when(cnt > W)
        def _():
            sub_flush(1, cnt, True, pend, True)

        @pl.when((cnt <= W) & ((pend & 2) > 0))
        def _():
            drain_flush(1)

    zero_acc(PBLK)
    plsc.subcore_barrier()
    run_pass_a()
    plsc.subcore_barrier()

    @pl.when(c == 0)
    def _():
        copy_out(P // CB, 0, out_pc)

    @pl.when(c == 1)
    def _():
        copy_out(P // CB, 0, out_pic)
    plsc.subcore_barrier()

    def phase_b_pass(q, _):
        base = q * SCHUNK
        zero_acc(SBLK)
        plsc.subcore_barrier()
        run_pass_b(base)
        plsc.subcore_barrier()

        @pl.when(c == 0)
        def _():
            copy_out(SCHUNK // CB, base, out_sc)

        @pl.when(c == 1)
        def _():
            copy_out(SCHUNK // CB, base, out_sic)
        plsc.subcore_barrier()
        return 0

    lax.fori_loop(0, NPASS, phase_b_pass, 0)


@jax.jit
def kernel(student_embeds, problem_embeds, a_rows, a_cols, a_values,
           ia_rows, ia_cols, ia_values):
    npad = EPAD - E
    pad_r = (jnp.arange(npad, dtype=jnp.int32) * 97) % S
    pad_c = (jnp.arange(npad, dtype=jnp.int32) * 89) % P
    pad_v = jnp.zeros((npad,), jnp.float32)

    def prep(x, pad):
        return jnp.concatenate([x, pad]).reshape(NCH, W)

    rows_h = jnp.concatenate([prep(a_rows, pad_r), prep(ia_rows, pad_r)])
    cols_h = jnp.concatenate([prep(a_cols, pad_c), prep(ia_cols, pad_c)])
    vals_h = jnp.concatenate([prep(a_values, pad_v), prep(ia_values, pad_v)])
    zeros_h = jnp.zeros((ZB, D), jnp.float32)

    mesh = plsc.VectorSubcoreMesh(core_axis_name="c", subcore_axis_name="s",
                                  num_cores=NC, num_subcores=NS)
    out = pl.kernel(
        _sc_body,
        out_type=(
            jax.ShapeDtypeStruct((S, D), jnp.float32),
            jax.ShapeDtypeStruct((S, D), jnp.float32),
            jax.ShapeDtypeStruct((P, D), jnp.float32),
            jax.ShapeDtypeStruct((P, D), jnp.float32),
        ),
        mesh=mesh,
        compiler_params=pltpu.CompilerParams(use_tc_tiling_on_sc=False,
                                             needs_layout_passes=False),
        scratch_types=[
            pltpu.VMEM_SHARED((ACCR, D), jnp.float32),
            pltpu.VMEM((ZB, D), jnp.float32),
            pltpu.VMEM((3, W, D), jnp.float32),
            pltpu.VMEM((4, W), jnp.int32),
            pltpu.VMEM((4, W), jnp.int32),
            pltpu.VMEM((4, W), jnp.float32),
            pltpu.VMEM((CCAP,), jnp.int32),
            pltpu.VMEM((CCAP,), jnp.int32),
            pltpu.VMEM((CCAP,), jnp.float32),
            pltpu.VMEM((W,), jnp.int32),
            pltpu.VMEM((W,), jnp.int32),
        ] + [pltpu.SemaphoreType.DMA] * 11,
    )(student_embeds, problem_embeds, rows_h, cols_h, vals_h, zeros_h)
    return out
